# Initial kernel scaffold; baseline (speedup 1.0000x reference)
import jax
import jax.numpy as jnp
from jax.experimental import pallas as pl


def kernel(x, l, k, params):
    raise NotImplementedError("write your pallas kernel here")


# speedup vs baseline: 1.7273x; 1.7273x over previous
import functools
import math

import jax
import jax.numpy as jnp
from jax.experimental import pallas as pl
from jax.experimental.pallas import tpu as pltpu

K = 40
ALPHA = 1.0 / math.sqrt(1.0 + 1e-5)


def _lrelu(x):
    return jnp.where(x >= 0, x, 0.2 * x)


def _knn_body(xt_tile_ref, xt_full_ref, idx_ref, val_ref, *, n, tn, kk):
    xt_tile = xt_tile_ref[0]
    xt_full = xt_full_ref[0]
    g = jax.lax.dot_general(
        xt_tile, xt_full, (((1,), (1,)), ((), ())),
        preferred_element_type=jnp.float32)
    nt = jnp.sum(xt_tile * xt_tile, axis=1, keepdims=True)
    nf = jnp.sum(xt_full * xt_full, axis=1, keepdims=True)
    pd = 2.0 * g - nt - nf.reshape(1, n)

    iota = jax.lax.broadcasted_iota(jnp.int32, (tn, n), 1)
    neg = jnp.float32(-jnp.inf)
    idx_cols = []
    val_cols = []
    for _ in range(kk):
        m = jnp.max(pd, axis=1, keepdims=True)
        cand = jnp.where(pd == m, iota, n)
        am = jnp.min(cand, axis=1, keepdims=True)
        idx_cols.append(am)
        val_cols.append(m)
        pd = jnp.where(iota == am, neg, pd)
    idx_ref[0] = jnp.concatenate(idx_cols, axis=1)
    val_ref[0] = jnp.concatenate(val_cols, axis=1)


def _knn(xt):
    b, n, c = xt.shape
    cp = 128
    xp = jnp.pad(xt, ((0, 0), (0, 0), (0, cp - c)))
    tn = 512
    grid = (b, n // tn)
    body = functools.partial(_knn_body, n=n, tn=tn, kk=K)
    idx, vals = pl.pallas_call(
        body,
        grid=grid,
        in_specs=[
            pl.BlockSpec((1, tn, cp), lambda bb, ii: (bb, ii, 0)),
            pl.BlockSpec((1, n, cp), lambda bb, ii: (bb, 0, 0)),
        ],
        out_specs=[
            pl.BlockSpec((1, tn, K), lambda bb, ii: (bb, ii, 0)),
            pl.BlockSpec((1, tn, K), lambda bb, ii: (bb, ii, 0)),
        ],
        out_shape=[
            jax.ShapeDtypeStruct((b, n, K), jnp.int32),
            jax.ShapeDtypeStruct((b, n, K), jnp.float32),
        ],
    )(xp, xp)
    return idx, vals


def _gather_rows(table_flat, gidx):
    return jnp.take(table_flat, gidx, axis=0)


def _attn_body(xj_ref, xi_ref, qq_ref, vw_ref, g2_ref, b2_ref, swp_ref,
               sc_ref, o_ref, *, tn, co, first):
    sq = jnp.sqrt(jnp.float32(1.0 + 1e-5))
    cin = xi_ref.shape[2]
    xj = xj_ref[0].reshape(tn, K, cin)
    xi = xi_ref[0]
    d1 = xj - xi[:, None, :]
    if first:
        d5 = jnp.sum(d1 * d1, axis=-1, keepdims=True)
        xib = jnp.broadcast_to(xi[:, None, :3], (tn, K, 3))
        f = (jnp.pad(d5, ((0, 0), (0, 0), (0, 127)))
             + jnp.pad(d1[:, :, :3], ((0, 0), (0, 0), (1, 124)))
             + jnp.pad(xib, ((0, 0), (0, 0), (4, 121)))
             + jnp.pad(xj[:, :, :3], ((0, 0), (0, 0), (7, 118))))
    else:
        xib = jnp.broadcast_to(xi[:, None, :], (tn, K, cin))
        f = jnp.concatenate([d1, xib], axis=-1)
    cf = f.shape[2]
    vpre = jax.lax.dot_general(
        f.reshape(tn * K, cf), vw_ref[...], (((1,), (0,)), ((), ())),
        preferred_element_type=jnp.float32).reshape(tn, K, co)
    v = _lrelu(vpre / sq * g2_ref[0][None, None, :] + b2_ref[0][None, None, :])
    fq = qq_ref[0][:, None, :] + v
    t128 = jax.lax.dot_general(
        fq.reshape(tn * K, co), swp_ref[...], (((1,), (0,)), ((), ())),
        preferred_element_type=jnp.float32).reshape(tn, K, 128)
    lane = jax.lax.broadcasted_iota(jnp.int32, (tn, K, 128), 2)
    t = jnp.sum(jnp.where(lane == 0, t128, 0.0), axis=-1)
    t = _lrelu(t / sq * sc_ref[0, 0] + sc_ref[0, 1])
    m = jnp.max(t, axis=1, keepdims=True)
    e = jnp.exp(t - m)
    s = e / jnp.sum(e, axis=1, keepdims=True)
    o_ref[0] = jnp.sum(v * s[:, :, None], axis=1)


def _attention(xjg, xtp, qq, vw, g2, b2, swp, sc, first):
    b, n, cin = xtp.shape
    co = qq.shape[2]
    cf = vw.shape[0]
    tn = 128
    grid = (b, n // tn)
    body = functools.partial(_attn_body, tn=tn, co=co, first=first)
    in_specs = [
        pl.BlockSpec((1, tn * K, cin), lambda bb, ii: (bb, ii, 0)),
        pl.BlockSpec((1, tn, cin), lambda bb, ii: (bb, ii, 0)),
        pl.BlockSpec((1, tn, co), lambda bb, ii: (bb, ii, 0)),
        pl.BlockSpec((cf, co), lambda bb, ii: (0, 0)),
        pl.BlockSpec((1, co), lambda bb, ii: (0, 0)),
        pl.BlockSpec((1, co), lambda bb, ii: (0, 0)),
        pl.BlockSpec((co, 128), lambda bb, ii: (0, 0)),
        pl.BlockSpec((1, co), lambda bb, ii: (0, 0)),
    ]
    return pl.pallas_call(
        body,
        grid=grid,
        in_specs=in_specs,
        out_specs=pl.BlockSpec((1, tn, co), lambda bb, ii: (bb, ii, 0)),
        out_shape=jax.ShapeDtypeStruct((b, n, co), jnp.float32),
    )(xjg, xtp, qq, vw, g2, b2, swp, sc)


def _pad_rows(w, cp):
    return jnp.pad(w, ((0, cp - w.shape[0]), (0, 0)))


def _pad_vec(v, cop):
    return jnp.pad(v, ((0, cop - v.shape[0]),))


def _lab_stage(p, xt, idx, firstlayer=False):
    b, n, c = xt.shape
    vw, kw = p['vw'], p['kw']
    co = vw.shape[1]
    cop = max(64, ((co + 63) // 64) * 64)
    if firstlayer:
        cin = 128
        vwp = jnp.pad(vw, ((0, 128 - vw.shape[0]), (0, cop - co)))
    else:
        cin = c
        vwp = jnp.pad(vw, ((0, 0), (0, cop - co)))
    sc = _pad_vec(jnp.concatenate([p['g3'], p['b3']]), cop).reshape(1, cop)
    qq = _lrelu((xt @ jnp.pad(kw, ((0, 0), (0, cop - co)))) / jnp.sqrt(1.0 + 1e-5)
                * _pad_vec(p['g1'], cop) + _pad_vec(p['b1'], cop))
    xtp = jnp.pad(xt, ((0, 0), (0, 0), (0, cin - c)))
    swp = jnp.pad(p['sw'], ((0, cop - co), (0, 127)))

    gidx = (idx + (jnp.arange(b, dtype=jnp.int32) * n)[:, None, None]).reshape(-1)
    xjg = _gather_rows(xtp.reshape(b * n, cin), gidx).reshape(b, n * K, cin)
    out = _attention(
        xjg, xtp, qq, vwp,
        _pad_vec(p['g2'], cop).reshape(1, cop),
        _pad_vec(p['b2'], cop).reshape(1, cop),
        swp, sc, firstlayer)
    return out[:, :, :co]


def _bn_v(x, g, b, axis):
    shape = [1] * x.ndim
    shape[axis] = x.shape[axis]
    return x / jnp.sqrt(1.0 + 1e-5) * g.reshape(shape) + b.reshape(shape)


def _gf(x, idx, firstlayer):
    b, c, n = x.shape
    xt = jnp.swapaxes(x, 1, 2)
    bidx = jnp.arange(b)[:, None, None]
    feature = xt[bidx, idx]
    x_glo = jnp.concatenate([jnp.sum(x, axis=1), jnp.mean(x, axis=1)], axis=-1)
    xr = jnp.broadcast_to(xt[:, :, None, :], (b, n, K, c))
    d1 = feature - xr
    if firstlayer:
        d5 = jnp.sum(d1 * d1, axis=-1, keepdims=True)
        f = jnp.concatenate([d5, d1, xr, feature], axis=3)
    else:
        f = jnp.concatenate([d1, xr], axis=3)
    f = jnp.transpose(f, (0, 3, 1, 2))
    return x[..., None], f, x_glo


def _conv2d(x, w):
    return jnp.einsum('bcnk,co->bonk', x, w)


def _conv1d(x, w):
    return jnp.einsum('bcn,co->bon', x, w)


def _lab_v(p, xq, xk):
    q = _lrelu(_bn_v(_conv2d(xq, p['kw']), p['g1'], p['b1'], 1))
    v = _lrelu(_bn_v(_conv2d(xk, p['vw']), p['g2'], p['b2'], 1))
    f = q + v
    s = _lrelu(_bn_v(_conv2d(f, p['sw']), p['g3'], p['b3'], 1))[:, 0]
    s = jax.nn.softmax(s, axis=2)
    return jnp.sum(v * s[:, None, :, :], axis=-1)


def _tnet_v(p, x):
    b = x.shape[0]
    h = _lrelu(_bn_v(_conv2d(x, p['c1']), p['g1'], p['b1'], 1))
    h = _lrelu(_bn_v(_conv2d(h, p['c2']), p['g2'], p['b2'], 1))
    h = jnp.max(h, axis=-1)
    h = _lrelu(_bn_v(_conv1d(h, p['c3']), p['g3'], p['b3'], 1))
    h = jnp.max(h, axis=-1)
    h = _lrelu(_bn_v(h @ p['l1'], p['g4'], p['b4'], 1))
    h = _lrelu(_bn_v(h @ p['l2'], p['g5'], p['b5'], 1))
    t = h @ p['tw'] + p['tb']
    return t.reshape(b, 3, 3)


def _gab_v(p, xq, xk):
    v = _lrelu(_bn_v(xk @ p['w1'], p['g1'], p['b1'], 2))
    if xq.shape[1] != 2048:
        xq = jnp.tile(xq, (1, 20))[:, :2048]
    f = v + xq[:, None, :]
    s = _lrelu(_bn_v(f @ p['w2'], p['g2'], p['b2'], 2))
    s = _lrelu(_bn_v(s @ p['w3'], p['g3'], p['b3'], 2))
    s = _lrelu(_bn_v(s @ p['w4'], p['g4'], p['b4'], 2))[..., 0]
    s = jax.nn.softmax(s, axis=1)
    return jnp.sum(v * s[..., None], axis=1)


def kernel(x, l, k, params):
    b = x.shape[0]
    n = x.shape[2]
    x = jnp.where(k == K, x, x)

    idx1, _ = _knn(jnp.swapaxes(x, 1, 2))
    _, f1, _ = _gf(x, idx1, False)
    t = _tnet_v(params['tn'], f1)
    x = jnp.swapaxes(jnp.einsum('bnc,bcd->bnd', jnp.swapaxes(x, 1, 2), t), 1, 2)

    idx2, _ = _knn(jnp.swapaxes(x, 1, 2))
    xv2, f2, xglo = _gf(x, idx2, True)
    x2 = _lab_v(params['lab1'], xv2, f2)
    idx3, _ = _knn(jnp.swapaxes(x2, 1, 2))
    xv3, f3, _ = _gf(x2, idx3, False)
    x3 = _lab_v(params['lab2'], xv3, f3)
    idx4, _ = _knn(jnp.swapaxes(x3, 1, 2))
    xv4, f4, _ = _gf(x3, idx4, False)
    x4 = _lab_v(params['lab3'], xv4, f4)
    idx5, _ = _knn(jnp.swapaxes(x4, 1, 2))
    xv5, f5, _ = _gf(x4, idx5, False)
    x5 = _lab_v(params['lab4'], xv5, f5)

    xc = jnp.concatenate([x2, x3, x4, x5], axis=1)
    g = _gab_v(params['gab'], xglo, jnp.swapaxes(xc, 1, 2))
    lf = _lrelu(_bn_v(_conv1d(l[:, :, None], params['c7']), params['g7'],
                      params['b7'], 1))
    gl = jnp.concatenate([g[:, :, None], lf], axis=1)
    gl = jnp.broadcast_to(gl, (b, gl.shape[1], n))
    h = jnp.concatenate([gl, xc], axis=1)
    h = _lrelu(_bn_v(_conv1d(h, params['c8']), params['g8'], params['b8'], 1))
    h = _lrelu(_bn_v(_conv1d(h, params['c9']), params['g9'], params['b9'], 1))
    h = _lrelu(_bn_v(_conv1d(h, params['c10']), params['g10'], params['b10'], 1))

    ht = jnp.swapaxes(h, 1, 2)
    idx6, _ = _knn(ht)
    h6 = _lab_stage(params['lab5'], ht, idx6)
    idx7, _ = _knn(h6)
    h7 = _lab_stage(params['lab6'], h6, idx7)
    idx8, _ = _knn(h7)
    h8 = _lab_stage(params['lab7'], h7, idx8)
    return jnp.swapaxes(h8, 1, 2)


# speedup vs baseline: 1.9020x; 1.1012x over previous
import functools
import math

import jax
import jax.numpy as jnp
from jax import lax
from jax.experimental import pallas as pl
from jax.experimental.pallas import tpu as pltpu
from jax.experimental.pallas import tpu_sc as plsc

K = 40
ALPHA = 1.0 / math.sqrt(1.0 + 1e-5)


def _lrelu(x):
    return jnp.where(x >= 0, x, 0.2 * x)


def _knn_body(xt_tile_ref, xt_full_ref, idx_ref, val_ref, *, n, tn, kk):
    xt_tile = xt_tile_ref[0]
    xt_full = xt_full_ref[0]
    g = jax.lax.dot_general(
        xt_tile, xt_full, (((1,), (1,)), ((), ())),
        preferred_element_type=jnp.float32)
    nt = jnp.sum(xt_tile * xt_tile, axis=1, keepdims=True)
    nf = jnp.sum(xt_full * xt_full, axis=1, keepdims=True)
    pd = 2.0 * g - nt - nf.reshape(1, n)

    iota = jax.lax.broadcasted_iota(jnp.int32, (tn, n), 1)
    neg = jnp.float32(-jnp.inf)
    idx_cols = []
    val_cols = []
    for _ in range(kk):
        m = jnp.max(pd, axis=1, keepdims=True)
        cand = jnp.where(pd == m, iota, n)
        am = jnp.min(cand, axis=1, keepdims=True)
        idx_cols.append(am)
        val_cols.append(m)
        pd = jnp.where(iota == am, neg, pd)
    idx_ref[0] = jnp.concatenate(idx_cols, axis=1)
    val_ref[0] = jnp.concatenate(val_cols, axis=1)


def _knn(xt):
    b, n, c = xt.shape
    cp = 128
    xp = jnp.pad(xt, ((0, 0), (0, 0), (0, cp - c)))
    tn = 512
    grid = (b, n // tn)
    body = functools.partial(_knn_body, n=n, tn=tn, kk=K)
    idx, vals = pl.pallas_call(
        body,
        grid=grid,
        in_specs=[
            pl.BlockSpec((1, tn, cp), lambda bb, ii: (bb, ii, 0)),
            pl.BlockSpec((1, n, cp), lambda bb, ii: (bb, 0, 0)),
        ],
        out_specs=[
            pl.BlockSpec((1, tn, K), lambda bb, ii: (bb, ii, 0)),
            pl.BlockSpec((1, tn, K), lambda bb, ii: (bb, ii, 0)),
        ],
        out_shape=[
            jax.ShapeDtypeStruct((b, n, K), jnp.int32),
            jax.ShapeDtypeStruct((b, n, K), jnp.float32),
        ],
    )(xp, xp)
    return idx, vals


def _gather_rows(table_flat, gidx):
    m = gidx.shape[0]
    d = table_flat.shape[1]
    info = plsc.get_sparse_core_info()
    nw = info.num_cores * info.num_subcores
    ch = 128
    b_per_w = m // nw
    nch = b_per_w // ch
    idx3 = gidx.reshape(nw, nch, ch)
    mesh = plsc.VectorSubcoreMesh(core_axis_name="c", subcore_axis_name="s")

    @functools.partial(
        pl.kernel, mesh=mesh,
        out_type=jax.ShapeDtypeStruct((m, d), jnp.float32),
        scratch_types=[
            pltpu.VMEM((nch, ch), jnp.int32),
            pltpu.VMEM((ch, d), jnp.float32),
            pltpu.SemaphoreType.DMA,
        ],
    )
    def k(table_hbm, idx_hbm, out_hbm, idx_v, rows_v, sem):
        wid = lax.axis_index("s") * info.num_cores + lax.axis_index("c")
        base = wid * b_per_w
        pltpu.sync_copy(idx_hbm.at[wid], idx_v)

        def body(j, carry):
            pltpu.async_copy(table_hbm.at[idx_v.at[j]], rows_v, sem).wait()
            pltpu.sync_copy(rows_v, out_hbm.at[pl.ds(base + j * ch, ch)])
            return carry

        lax.fori_loop(0, nch, body, 0)

    return k(table_flat, idx3)


def _attn_body(xj_ref, xi_ref, qq_ref, vw_ref, g2_ref, b2_ref, swp_ref,
               sc_ref, o_ref, *, tn, co, first):
    sq = jnp.sqrt(jnp.float32(1.0 + 1e-5))
    cin = xi_ref.shape[2]
    xj = xj_ref[0].reshape(tn, K, cin)
    xi = xi_ref[0]
    d1 = xj - xi[:, None, :]
    if first:
        d5 = jnp.sum(d1 * d1, axis=-1, keepdims=True)
        xib = jnp.broadcast_to(xi[:, None, :3], (tn, K, 3))
        f = (jnp.pad(d5, ((0, 0), (0, 0), (0, 127)))
             + jnp.pad(d1[:, :, :3], ((0, 0), (0, 0), (1, 124)))
             + jnp.pad(xib, ((0, 0), (0, 0), (4, 121)))
             + jnp.pad(xj[:, :, :3], ((0, 0), (0, 0), (7, 118))))
    else:
        xib = jnp.broadcast_to(xi[:, None, :], (tn, K, cin))
        f = jnp.concatenate([d1, xib], axis=-1)
    cf = f.shape[2]
    vpre = jax.lax.dot_general(
        f.reshape(tn * K, cf), vw_ref[...], (((1,), (0,)), ((), ())),
        preferred_element_type=jnp.float32).reshape(tn, K, co)
    v = _lrelu(vpre / sq * g2_ref[0][None, None, :] + b2_ref[0][None, None, :])
    fq = qq_ref[0][:, None, :] + v
    t128 = jax.lax.dot_general(
        fq.reshape(tn * K, co), swp_ref[...], (((1,), (0,)), ((), ())),
        preferred_element_type=jnp.float32).reshape(tn, K, 128)
    lane = jax.lax.broadcasted_iota(jnp.int32, (tn, K, 128), 2)
    t = jnp.sum(jnp.where(lane == 0, t128, 0.0), axis=-1)
    t = _lrelu(t / sq * sc_ref[0, 0] + sc_ref[0, 1])
    m = jnp.max(t, axis=1, keepdims=True)
    e = jnp.exp(t - m)
    s = e / jnp.sum(e, axis=1, keepdims=True)
    o_ref[0] = jnp.sum(v * s[:, :, None], axis=1)


def _attention(xjg, xtp, qq, vw, g2, b2, swp, sc, first):
    b, n, cin = xtp.shape
    co = qq.shape[2]
    cf = vw.shape[0]
    tn = 128
    grid = (b, n // tn)
    body = functools.partial(_attn_body, tn=tn, co=co, first=first)
    in_specs = [
        pl.BlockSpec((1, tn * K, cin), lambda bb, ii: (bb, ii, 0)),
        pl.BlockSpec((1, tn, cin), lambda bb, ii: (bb, ii, 0)),
        pl.BlockSpec((1, tn, co), lambda bb, ii: (bb, ii, 0)),
        pl.BlockSpec((cf, co), lambda bb, ii: (0, 0)),
        pl.BlockSpec((1, co), lambda bb, ii: (0, 0)),
        pl.BlockSpec((1, co), lambda bb, ii: (0, 0)),
        pl.BlockSpec((co, 128), lambda bb, ii: (0, 0)),
        pl.BlockSpec((1, co), lambda bb, ii: (0, 0)),
    ]
    return pl.pallas_call(
        body,
        grid=grid,
        in_specs=in_specs,
        out_specs=pl.BlockSpec((1, tn, co), lambda bb, ii: (bb, ii, 0)),
        out_shape=jax.ShapeDtypeStruct((b, n, co), jnp.float32),
    )(xjg, xtp, qq, vw, g2, b2, swp, sc)


def _pad_rows(w, cp):
    return jnp.pad(w, ((0, cp - w.shape[0]), (0, 0)))


def _pad_vec(v, cop):
    return jnp.pad(v, ((0, cop - v.shape[0]),))


def _lab_stage(p, xt, idx, firstlayer=False):
    b, n, c = xt.shape
    vw, kw = p['vw'], p['kw']
    co = vw.shape[1]
    cop = max(64, ((co + 63) // 64) * 64)
    cin = 128
    if firstlayer:
        vwp = jnp.pad(vw, ((0, 128 - vw.shape[0]), (0, cop - co)))
    else:
        vwp = jnp.concatenate([
            jnp.pad(vw[:c], ((0, 128 - c), (0, cop - co))),
            jnp.pad(vw[c:], ((0, 128 - c), (0, cop - co))),
        ])
    sc = _pad_vec(jnp.concatenate([p['g3'], p['b3']]), cop).reshape(1, cop)
    qq = _lrelu((xt @ jnp.pad(kw, ((0, 0), (0, cop - co)))) / jnp.sqrt(1.0 + 1e-5)
                * _pad_vec(p['g1'], cop) + _pad_vec(p['b1'], cop))
    xtp = jnp.pad(xt, ((0, 0), (0, 0), (0, cin - c)))
    swp = jnp.pad(p['sw'], ((0, cop - co), (0, 127)))

    gidx = (idx + (jnp.arange(b, dtype=jnp.int32) * n)[:, None, None]).reshape(-1)
    xjg = _gather_rows(xtp.reshape(b * n, cin), gidx).reshape(b, n * K, cin)
    out = _attention(
        xjg, xtp, qq, vwp,
        _pad_vec(p['g2'], cop).reshape(1, cop),
        _pad_vec(p['b2'], cop).reshape(1, cop),
        swp, sc, firstlayer)
    return out[:, :, :co]


def _bn_v(x, g, b, axis):
    shape = [1] * x.ndim
    shape[axis] = x.shape[axis]
    return x / jnp.sqrt(1.0 + 1e-5) * g.reshape(shape) + b.reshape(shape)


def _gf(x, idx, firstlayer):
    b, c, n = x.shape
    xt = jnp.swapaxes(x, 1, 2)
    bidx = jnp.arange(b)[:, None, None]
    feature = xt[bidx, idx]
    x_glo = jnp.concatenate([jnp.sum(x, axis=1), jnp.mean(x, axis=1)], axis=-1)
    xr = jnp.broadcast_to(xt[:, :, None, :], (b, n, K, c))
    d1 = feature - xr
    if firstlayer:
        d5 = jnp.sum(d1 * d1, axis=-1, keepdims=True)
        f = jnp.concatenate([d5, d1, xr, feature], axis=3)
    else:
        f = jnp.concatenate([d1, xr], axis=3)
    f = jnp.transpose(f, (0, 3, 1, 2))
    return x[..., None], f, x_glo


def _conv2d(x, w):
    return jnp.einsum('bcnk,co->bonk', x, w)


def _conv1d(x, w):
    return jnp.einsum('bcn,co->bon', x, w)


def _lab_v(p, xq, xk):
    q = _lrelu(_bn_v(_conv2d(xq, p['kw']), p['g1'], p['b1'], 1))
    v = _lrelu(_bn_v(_conv2d(xk, p['vw']), p['g2'], p['b2'], 1))
    f = q + v
    s = _lrelu(_bn_v(_conv2d(f, p['sw']), p['g3'], p['b3'], 1))[:, 0]
    s = jax.nn.softmax(s, axis=2)
    return jnp.sum(v * s[:, None, :, :], axis=-1)


def _tnet_v(p, x):
    b = x.shape[0]
    h = _lrelu(_bn_v(_conv2d(x, p['c1']), p['g1'], p['b1'], 1))
    h = _lrelu(_bn_v(_conv2d(h, p['c2']), p['g2'], p['b2'], 1))
    h = jnp.max(h, axis=-1)
    h = _lrelu(_bn_v(_conv1d(h, p['c3']), p['g3'], p['b3'], 1))
    h = jnp.max(h, axis=-1)
    h = _lrelu(_bn_v(h @ p['l1'], p['g4'], p['b4'], 1))
    h = _lrelu(_bn_v(h @ p['l2'], p['g5'], p['b5'], 1))
    t = h @ p['tw'] + p['tb']
    return t.reshape(b, 3, 3)


def _gab_v(p, xq, xk):
    v = _lrelu(_bn_v(xk @ p['w1'], p['g1'], p['b1'], 2))
    if xq.shape[1] != 2048:
        xq = jnp.tile(xq, (1, 20))[:, :2048]
    f = v + xq[:, None, :]
    s = _lrelu(_bn_v(f @ p['w2'], p['g2'], p['b2'], 2))
    s = _lrelu(_bn_v(s @ p['w3'], p['g3'], p['b3'], 2))
    s = _lrelu(_bn_v(s @ p['w4'], p['g4'], p['b4'], 2))[..., 0]
    s = jax.nn.softmax(s, axis=1)
    return jnp.sum(v * s[..., None], axis=1)


def kernel(x, l, k, params):
    b = x.shape[0]
    n = x.shape[2]
    x = jnp.where(k == K, x, x)

    idx1, _ = _knn(jnp.swapaxes(x, 1, 2))
    _, f1, _ = _gf(x, idx1, False)
    t = _tnet_v(params['tn'], f1)
    x = jnp.swapaxes(jnp.einsum('bnc,bcd->bnd', jnp.swapaxes(x, 1, 2), t), 1, 2)

    idx2, _ = _knn(jnp.swapaxes(x, 1, 2))
    xv2, f2, xglo = _gf(x, idx2, True)
    x2 = _lab_v(params['lab1'], xv2, f2)
    idx3, _ = _knn(jnp.swapaxes(x2, 1, 2))
    xv3, f3, _ = _gf(x2, idx3, False)
    x3 = _lab_v(params['lab2'], xv3, f3)
    idx4, _ = _knn(jnp.swapaxes(x3, 1, 2))
    xv4, f4, _ = _gf(x3, idx4, False)
    x4 = _lab_v(params['lab3'], xv4, f4)
    idx5, _ = _knn(jnp.swapaxes(x4, 1, 2))
    xv5, f5, _ = _gf(x4, idx5, False)
    x5 = _lab_v(params['lab4'], xv5, f5)

    xc = jnp.concatenate([x2, x3, x4, x5], axis=1)
    g = _gab_v(params['gab'], xglo, jnp.swapaxes(xc, 1, 2))
    lf = _lrelu(_bn_v(_conv1d(l[:, :, None], params['c7']), params['g7'],
                      params['b7'], 1))
    gl = jnp.concatenate([g[:, :, None], lf], axis=1)
    gl = jnp.broadcast_to(gl, (b, gl.shape[1], n))
    h = jnp.concatenate([gl, xc], axis=1)
    h = _lrelu(_bn_v(_conv1d(h, params['c8']), params['g8'], params['b8'], 1))
    h = _lrelu(_bn_v(_conv1d(h, params['c9']), params['g9'], params['b9'], 1))
    h = _lrelu(_bn_v(_conv1d(h, params['c10']), params['g10'], params['b10'], 1))

    ht = jnp.swapaxes(h, 1, 2)
    idx6, _ = _knn(ht)
    h6 = _lab_stage(params['lab5'], ht, idx6)
    idx7, _ = _knn(h6)
    h7 = _lab_stage(params['lab6'], h6, idx7)
    idx8, _ = _knn(h7)
    h8 = _lab_stage(params['lab7'], h7, idx8)
    return jnp.swapaxes(h8, 1, 2)
